# Initial kernel scaffold; baseline (speedup 1.0000x reference)
import jax
import jax.numpy as jnp
from jax.experimental import pallas as pl


def kernel(value, depth, position, src_table, depth_table, sp_table0, sp_table1, sp_table2):
    raise NotImplementedError("write your pallas kernel here")


# speedup vs baseline: 1.9407x; 1.9407x over previous
import functools

import jax
import jax.numpy as jnp
from jax import lax
from jax.experimental import pallas as pl
from jax.experimental.pallas import tpu as pltpu
from jax.experimental.pallas import tpu_sc as plsc

NC = 2
NS = 16
NW = NC * NS
LANES = 16

B, L = 4, 8192
N = B * L
TOK_PER_W = N // NW
T = 64
NCHUNK = TOK_PER_W // T
D = 256


def _sc_body(vi, di, p0i, p1i, p2i, src_t, dep_t, sp0_t, sp1_t, sp2_t,
             out_hbm,
             vi_v, di_v, p0_v, p1_v, p2_v,
             b0, b1, b2, b3, b4,
             s0, s1, s2, s3, s4):
    wid = lax.axis_index("s") * NC + lax.axis_index("c")
    base = wid * TOK_PER_W

    pltpu.sync_copy(vi.at[wid], vi_v)
    pltpu.sync_copy(di.at[wid], di_v)
    pltpu.sync_copy(p0i.at[wid], p0_v)
    pltpu.sync_copy(p1i.at[wid], p1_v)
    pltpu.sync_copy(p2i.at[wid], p2_v)

    def chunk(c, carry):
        c0 = pltpu.async_copy(src_t.at[vi_v.at[c]], b0, s0)
        c1 = pltpu.async_copy(dep_t.at[di_v.at[c]], b1, s1)
        c2 = pltpu.async_copy(sp0_t.at[p0_v.at[c]], b2, s2)
        c3 = pltpu.async_copy(sp1_t.at[p1_v.at[c]], b3, s3)
        c4 = pltpu.async_copy(sp2_t.at[p2_v.at[c]], b4, s4)
        c0.wait(); c1.wait(); c2.wait(); c3.wait(); c4.wait()

        def row(r, carry2):
            for d in range(D // LANES):
                sl = pl.ds(d * LANES, LANES)
                b0[r, sl] = (b0[r, sl] + b1[r, sl]) + (b2[r, sl] + b3[r, sl]) + b4[r, sl]
            return carry2

        lax.fori_loop(0, T, row, 0, unroll=False)
        pltpu.sync_copy(b0, out_hbm.at[pl.ds(base + c * T, T)])
        return carry

    lax.fori_loop(0, NCHUNK, chunk, 0, unroll=False)


@jax.jit
def _embed_sum(vi, di, p0i, p1i, p2i, src_t, dep_t, sp0_t, sp1_t, sp2_t):
    kern = pl.kernel(
        _sc_body,
        out_type=jax.ShapeDtypeStruct((N, D), jnp.float32),
        mesh=plsc.VectorSubcoreMesh(
            core_axis_name="c", subcore_axis_name="s",
            num_cores=NC, num_subcores=NS),
        scratch_types=(
            [pltpu.VMEM((NCHUNK, T), jnp.int32)] * 5
            + [pltpu.VMEM((T, D), jnp.float32)] * 5
            + [pltpu.SemaphoreType.DMA] * 5
        ),
    )
    return kern(vi, di, p0i, p1i, p2i, src_t, dep_t, sp0_t, sp1_t, sp2_t)


def kernel(value, depth, position, src_table, depth_table, sp_table0,
           sp_table1, sp_table2):
    shp = (NW, NCHUNK, T)
    vi = value.reshape(shp).astype(jnp.int32)
    di = depth.reshape(shp).astype(jnp.int32)
    p0i = position[:, :, 0].reshape(shp).astype(jnp.int32)
    p1i = position[:, :, 1].reshape(shp).astype(jnp.int32)
    p2i = position[:, :, 2].reshape(shp).astype(jnp.int32)
    out = _embed_sum(vi, di, p0i, p1i, p2i, src_table, depth_table,
                     sp_table0, sp_table1, sp_table2)
    return out.reshape(B, L, D)


# speedup vs baseline: 1.9749x; 1.0176x over previous
import jax
import jax.numpy as jnp
from jax import lax
from jax.experimental import pallas as pl
from jax.experimental.pallas import tpu as pltpu
from jax.experimental.pallas import tpu_sc as plsc

NC = 2
NS = 16
NW = NC * NS
LANES = 16

B, L = 4, 8192
N = B * L
TOK_PER_W = N // NW
T = 32
NCHUNK = TOK_PER_W // T
HALF = NCHUNK // 2
D = 256


def _sc_body(vi, di, p0i, p1i, p2i, src_t, dep_t, sp0_t, sp1_t, sp2_t,
             out_hbm,
             vi_v, di_v, p0_v, p1_v, p2_v,
             a0, a1, a2, a3, a4,
             b0, b1, b2, b3, b4,
             oa, ob,
             sa0, sa1, sa2, sa3, sa4,
             sb0, sb1, sb2, sb3, sb4,
             soa, sob):
    wid = lax.axis_index("s") * NC + lax.axis_index("c")
    base = wid * TOK_PER_W

    pltpu.sync_copy(vi.at[wid], vi_v)
    pltpu.sync_copy(di.at[wid], di_v)
    pltpu.sync_copy(p0i.at[wid], p0_v)
    pltpu.sync_copy(p1i.at[wid], p1_v)
    pltpu.sync_copy(p2i.at[wid], p2_v)

    sets = (
        ((a0, a1, a2, a3, a4), (sa0, sa1, sa2, sa3, sa4), oa, soa),
        ((b0, b1, b2, b3, b4), (sb0, sb1, sb2, sb3, sb4), ob, sob),
    )

    def gathers(c, p):
        bufs, sems, _, _ = sets[p]
        return (
            pltpu.make_async_copy(src_t.at[vi_v.at[c]], bufs[0], sems[0]),
            pltpu.make_async_copy(dep_t.at[di_v.at[c]], bufs[1], sems[1]),
            pltpu.make_async_copy(sp0_t.at[p0_v.at[c]], bufs[2], sems[2]),
            pltpu.make_async_copy(sp1_t.at[p1_v.at[c]], bufs[3], sems[3]),
            pltpu.make_async_copy(sp2_t.at[p2_v.at[c]], bufs[4], sems[4]),
        )

    def fire(c, p):
        for d in gathers(c, p):
            d.start()

    def wait_gathers(c, p):
        for d in gathers(c, p):
            d.wait()

    def out_copy(c, p):
        _, _, obuf, osem = sets[p]
        return pltpu.make_async_copy(
            obuf, out_hbm.at[pl.ds(base + c * T, T)], osem)

    def process(c, p, k):
        bufs, _, obuf, _ = sets[p]
        wait_gathers(c, p)

        @pl.when(k > 0)
        def _():
            out_copy(c - 2, p).wait()

        g0, g1, g2, g3, g4 = bufs

        def row(r, carry):
            for d in range(D // LANES):
                sl = pl.ds(d * LANES, LANES)
                obuf[r, sl] = ((g0[r, sl] + g1[r, sl])
                               + (g2[r, sl] + g3[r, sl])) + g4[r, sl]
            return carry

        lax.fori_loop(0, T, row, 0, unroll=False)
        out_copy(c, p).start()

    fire(0, 0)

    def pair(k, carry):
        c0 = 2 * k
        fire(c0 + 1, 1)
        process(c0, 0, k)

        @pl.when(k < HALF - 1)
        def _():
            fire(c0 + 2, 0)

        process(c0 + 1, 1, k)
        return carry

    lax.fori_loop(0, HALF, pair, 0, unroll=False)
    out_copy(NCHUNK - 2, 0).wait()
    out_copy(NCHUNK - 1, 1).wait()


@jax.jit
def _embed_sum(vi, di, p0i, p1i, p2i, src_t, dep_t, sp0_t, sp1_t, sp2_t):
    kern = pl.kernel(
        _sc_body,
        out_type=jax.ShapeDtypeStruct((N, D), jnp.float32),
        mesh=plsc.VectorSubcoreMesh(
            core_axis_name="c", subcore_axis_name="s",
            num_cores=NC, num_subcores=NS),
        scratch_types=(
            [pltpu.VMEM((NCHUNK, T), jnp.int32)] * 5
            + [pltpu.VMEM((T, D), jnp.float32)] * 12
            + [pltpu.SemaphoreType.DMA] * 12
        ),
    )
    return kern(vi, di, p0i, p1i, p2i, src_t, dep_t, sp0_t, sp1_t, sp2_t)


def kernel(value, depth, position, src_table, depth_table, sp_table0,
           sp_table1, sp_table2):
    shp = (NW, NCHUNK, T)
    vi = value.reshape(shp).astype(jnp.int32)
    di = depth.reshape(shp).astype(jnp.int32)
    p0i = position[:, :, 0].reshape(shp).astype(jnp.int32)
    p1i = position[:, :, 1].reshape(shp).astype(jnp.int32)
    p2i = position[:, :, 2].reshape(shp).astype(jnp.int32)
    out = _embed_sum(vi, di, p0i, p1i, p2i, src_table, depth_table,
                     sp_table0, sp_table1, sp_table2)
    return out.reshape(B, L, D)
